# baseline (speedup 1.0000x reference)
import jax
import jax.numpy as jnp
from jax import lax
from jax.experimental import pallas as pl
from jax.experimental.pallas import tpu as pltpu
from jax.experimental.pallas import tpu_sc as plsc

N_NODES = 50000
D = 512
B = 8192
MAX_POS = 50
MAX_NEG = 50
S = 10

NC, NS, L = 2, 16, 16
NW = NC * NS
NB = B // NW
CH = 4
NCH = NB // CH
SEG = 88
NJ = D // L
NKI = (NB * S) // 128

_RP = (31, 5, 40, 13, 47, 18, 43, 36, 22, 17)
_RN = (48, 0, 23, 39, 16, 10, 3, 24, 35, 11)


def _sc_body(nb_hbm, emb_hbm, pt_hbm, nt_hbm, out_hbm,
             nodes_v, tidx, pv2, nv2, aidx, buf0, buf1,
             accT, scores_v, semp, sem0, sem1):
    wid = lax.axis_index("c") * NS + lax.axis_index("s")
    base = wid * NB
    pltpu.sync_copy(nb_hbm.at[pl.ds(base, NB)], nodes_v)
    lane = lax.iota(jnp.int32, 16)

    for j in range(NB // L):
        nv = nodes_v[pl.ds(j * L, L)]
        n = lane + j * L
        for si in range(S):
            plsc.store_scatter(tidx, [si * NB + n], nv + si * N_NODES)

    def _sid_copies():
        for k in range(NKI):
            isl = tidx.at[pl.ds(k * 128, 128)]
            dst = pl.ds(k * 128, 128)
            yield (pt_hbm.at[isl], pv2.at[dst])
            yield (nt_hbm.at[isl], nv2.at[dst])

    for src, dst in _sid_copies():
        pltpu.async_copy(src, dst, semp)
    for src, dst in _sid_copies():
        pltpu.make_async_copy(src, dst, semp).wait()

    def _rearrange(j, carry):
        n = lane + j * L
        c = n >> 2
        q = n & 3
        bcol = c * SEG + q * S
        for si in range(S):
            plsc.store_scatter(aidx, [bcol + si],
                               pv2[pl.ds(si * NB + j * L, L)])
            plsc.store_scatter(aidx, [bcol + CH * S + si],
                               nv2[pl.ds(si * NB + j * L, L)])
        nv = nodes_v[pl.ds(j * L, L)]
        plsc.store_scatter(aidx, [c * SEG + 80 + q], nv)
        plsc.store_scatter(aidx, [c * SEG + 84 + q], nv)
        return carry

    lax.fori_loop(0, NB // L, _rearrange, 0)

    zero = jnp.zeros((L,), jnp.float32)
    for l in range(16):
        accT[pl.ds(l * 32, 16)] = zero
        accT[pl.ds(l * 32 + 16, 16)] = zero

    def issue(c, buf, sem):
        pltpu.async_copy(emb_hbm.at[aidx.at[pl.ds(c * SEG, SEG)]], buf, sem)

    def wait_(c, buf, sem):
        pltpu.make_async_copy(emb_hbm.at[aidx.at[pl.ds(c * SEG, SEG)]], buf, sem).wait()

    def compute(c, buf):
        for q in range(CH):
            nrow = 80 + q
            rows = [q * S + t for t in range(S)] + [40 + q * S + t for t in range(S)]

            def body(j, accs):
                o = j * (2 * L)
                nvec0 = buf[nrow, pl.ds(o, L)]
                nvec1 = buf[nrow, pl.ds(o + L, L)]
                return tuple(
                    a + nvec0 * buf[r, pl.ds(o, L)]
                      + nvec1 * buf[r, pl.ds(o + L, L)]
                    for a, r in zip(accs, rows))

            accs = lax.fori_loop(
                0, NJ // 2, body,
                tuple(jnp.zeros((L,), jnp.float32) for _ in range(2 * S)))
            for t in range(2 * S):
                col = t if t < S else L + (t - S)
                plsc.store_scatter(accT, [lane * 32 + col], accs[t])

            def tree(vs):
                while len(vs) > 1:
                    vs = [a + b for a, b in zip(vs[::2], vs[1::2])]
                return vs[0]

            g = c * CH + q
            scores_v[g >> 2, pl.ds((g & 3) * 32, 16)] = tree(
                [accT[pl.ds(l * 32, 16)] for l in range(16)])
            scores_v[g >> 2, pl.ds((g & 3) * 32 + 16, 16)] = tree(
                [accT[pl.ds(l * 32 + 16, 16)] for l in range(16)])

    issue(0, buf0, sem0)

    def iter_body(i, carry):
        c0 = 2 * i
        c1 = c0 + 1
        issue(c1, buf1, sem1)
        wait_(c0, buf0, sem0)
        compute(c0, buf0)

        @pl.when(i < NCH // 2 - 1)
        def _():
            issue(c0 + 2, buf0, sem0)

        wait_(c1, buf1, sem1)
        compute(c1, buf1)
        return carry

    lax.fori_loop(0, NCH // 2, iter_body, 0)
    pltpu.sync_copy(scores_v, out_hbm.at[pl.ds(wid * (NB * 32 // 128), NB * 32 // 128)])


_sc_scores = pl.kernel(
    _sc_body,
    out_type=jax.ShapeDtypeStruct((B * 32 // 128, 128), jnp.float32),
    mesh=plsc.VectorSubcoreMesh(core_axis_name="c", subcore_axis_name="s"),
    compiler_params=pltpu.CompilerParams(needs_layout_passes=False),
    scratch_types=[
        pltpu.VMEM((NB,), jnp.int32),
        pltpu.VMEM((NB * S,), jnp.int32),
        pltpu.VMEM((NB * S,), jnp.int32),
        pltpu.VMEM((NB * S,), jnp.int32),
        pltpu.VMEM((NCH * SEG,), jnp.int32),
        pltpu.VMEM((SEG, D), jnp.float32),
        pltpu.VMEM((SEG, D), jnp.float32),
        pltpu.VMEM((16 * 32,), jnp.float32),
        pltpu.VMEM((NB * 32 // 128, 128), jnp.float32),
        pltpu.SemaphoreType.DMA,
        pltpu.SemaphoreType.DMA,
        pltpu.SemaphoreType.DMA,
    ],
)


def _sel_body(pt_ref, nt_ref, po_ref, no_ref):
    ci = lax.broadcasted_iota(jnp.int32, (16, MAX_POS), 1)
    ri = lax.broadcasted_iota(jnp.int32, (16, MAX_POS), 0)
    dn = (((1,), (1,)), ((), ()))
    for ref, cols, out in ((pt_ref, _RP, po_ref), (nt_ref, _RN, no_ref)):
        xi = ref[...]
        tgt = jnp.where(ri == 0, cols[0], 0)
        for s_ in range(1, S):
            tgt = tgt + jnp.where(ri == s_, cols[s_], 0)
        sel_mat = jnp.where(ci == tgt, 1.0, 0.0)
        hi = lax.dot_general(sel_mat, (xi >> 8).astype(jnp.float32), dn,
                             preferred_element_type=jnp.float32)
        lo = lax.dot_general(sel_mat, (xi & 255).astype(jnp.float32), dn,
                             preferred_element_type=jnp.float32)
        out[...] = ((hi + 0.25).astype(jnp.int32) * 256
                    + (lo + 0.25).astype(jnp.int32))


_SELBLK = 4096

_sel = pl.pallas_call(
    _sel_body,
    grid=(pl.cdiv(N_NODES, _SELBLK),),
    in_specs=[pl.BlockSpec((_SELBLK, MAX_POS), lambda i: (i, 0)),
              pl.BlockSpec((_SELBLK, MAX_NEG), lambda i: (i, 0))],
    out_specs=(pl.BlockSpec((16, _SELBLK), lambda i: (0, i)),
               pl.BlockSpec((16, _SELBLK), lambda i: (0, i))),
    out_shape=(jax.ShapeDtypeStruct((16, N_NODES), jnp.int32),
               jax.ShapeDtypeStruct((16, N_NODES), jnp.int32)),
)


def _loss_body(s_ref, o_ref):
    s = s_ref[...]
    col = lax.broadcasted_iota(jnp.int32, s.shape, 1) % 32
    pos_m = col < S
    neg_m = (col >= L) & (col < L + S)
    sp = jnp.log(1.0 + jnp.exp(-jnp.abs(s)))
    softplus_neg_s = jnp.maximum(-s, 0.0) + sp
    softplus_pos_s = jnp.maximum(s, 0.0) + sp
    tot = (jnp.sum(jnp.where(pos_m, softplus_neg_s, 0.0))
           + float(S) * jnp.sum(jnp.where(neg_m, softplus_pos_s, 0.0)))
    o_ref[...] = jnp.reshape(tot / float(B), (1, 1))


_loss = pl.pallas_call(
    _loss_body,
    out_shape=jax.ShapeDtypeStruct((1, 1), jnp.float32),
)


def kernel(node_batch, embeddings, pos_table, neg_table):
    psel, nsel = _sel(pos_table, neg_table)
    scores = _sc_scores(node_batch, embeddings,
                        psel.reshape(-1), nsel.reshape(-1))
    return _loss(scores)[0, 0]


# speedup vs baseline: 1.0262x; 1.0262x over previous
import jax
import jax.numpy as jnp
from jax import lax
from jax.experimental import pallas as pl
from jax.experimental.pallas import tpu as pltpu
from jax.experimental.pallas import tpu_sc as plsc

N_NODES = 50000
D = 512
B = 8192
MAX_POS = 50
MAX_NEG = 50
S = 10

NC, NS, L = 2, 16, 16
NW = NC * NS
NB = B // NW
CH = 4
NCH = NB // CH
SEG = 88
NJ = D // L
NKI = (NB * S) // 128

_RP = (31, 5, 40, 13, 47, 18, 43, 36, 22, 17)
_RN = (48, 0, 23, 39, 16, 10, 3, 24, 35, 11)


def _sc_body(nb_hbm, emb_hbm, pt_hbm, nt_hbm, out_hbm,
             nodes_v, tidx, pv2, nv2, aidx, buf0, buf1,
             accT, scores_v, semp, sem0, sem1):
    wid = lax.axis_index("c") * NS + lax.axis_index("s")
    base = wid * NB
    pltpu.sync_copy(nb_hbm.at[pl.ds(base, NB)], nodes_v)
    lane = lax.iota(jnp.int32, 16)

    for j in range(NB // L):
        nv = nodes_v[pl.ds(j * L, L)]
        n = lane + j * L
        for si in range(S):
            plsc.store_scatter(tidx, [si * NB + n], nv + si * N_NODES)

    def _sid_copies():
        for k in range(NKI):
            isl = tidx.at[pl.ds(k * 128, 128)]
            dst = pl.ds(k * 128, 128)
            yield (pt_hbm.at[isl], pv2.at[dst])
            yield (nt_hbm.at[isl], nv2.at[dst])

    for src, dst in _sid_copies():
        pltpu.async_copy(src, dst, semp)
    for src, dst in _sid_copies():
        pltpu.make_async_copy(src, dst, semp).wait()

    def _rearrange(j, carry):
        n = lane + j * L
        c = n >> 2
        q = n & 3
        bcol = c * SEG + q * S
        for si in range(S):
            plsc.store_scatter(aidx, [bcol + si],
                               pv2[pl.ds(si * NB + j * L, L)])
            plsc.store_scatter(aidx, [bcol + CH * S + si],
                               nv2[pl.ds(si * NB + j * L, L)])
        nv = nodes_v[pl.ds(j * L, L)]
        plsc.store_scatter(aidx, [c * SEG + 80 + q], nv)
        plsc.store_scatter(aidx, [c * SEG + 84 + q], nv)
        return carry

    lax.fori_loop(0, NB // L, _rearrange, 0)

    zero = jnp.zeros((L,), jnp.float32)
    for l in range(16):
        accT[pl.ds(l * 32, 16)] = zero
        accT[pl.ds(l * 32 + 16, 16)] = zero

    def issue(c, buf, sem):
        pltpu.async_copy(emb_hbm.at[aidx.at[pl.ds(c * SEG, SEG)]], buf, sem)

    def wait_(c, buf, sem):
        pltpu.make_async_copy(emb_hbm.at[aidx.at[pl.ds(c * SEG, SEG)]], buf, sem).wait()

    def compute(c, buf):
        for q in range(CH):
            nrow = 80 + q
            rows = [q * S + t for t in range(S)] + [40 + q * S + t for t in range(S)]

            def body(j, accs):
                o = j * L
                nvec = buf[nrow, pl.ds(o, L)]
                return tuple(a + nvec * buf[r, pl.ds(o, L)]
                             for a, r in zip(accs, rows))

            accs = lax.fori_loop(
                0, NJ, body,
                tuple(jnp.zeros((L,), jnp.float32) for _ in range(2 * S)))
            for t in range(2 * S):
                col = t if t < S else L + (t - S)
                plsc.store_scatter(accT, [lane * 32 + col], accs[t])

            def tree(vs):
                while len(vs) > 1:
                    vs = [a + b for a, b in zip(vs[::2], vs[1::2])]
                return vs[0]

            g = c * CH + q
            scores_v[g >> 2, pl.ds((g & 3) * 32, 16)] = tree(
                [accT[pl.ds(l * 32, 16)] for l in range(16)])
            scores_v[g >> 2, pl.ds((g & 3) * 32 + 16, 16)] = tree(
                [accT[pl.ds(l * 32 + 16, 16)] for l in range(16)])

    issue(0, buf0, sem0)

    def iter_body(i, carry):
        c0 = 2 * i
        c1 = c0 + 1
        issue(c1, buf1, sem1)
        wait_(c0, buf0, sem0)
        compute(c0, buf0)

        @pl.when(i < NCH // 2 - 1)
        def _():
            issue(c0 + 2, buf0, sem0)

        wait_(c1, buf1, sem1)
        compute(c1, buf1)
        return carry

    lax.fori_loop(0, NCH // 2, iter_body, 0)
    pltpu.sync_copy(scores_v, out_hbm.at[pl.ds(wid * (NB * 32 // 128), NB * 32 // 128)])


_sc_scores = pl.kernel(
    _sc_body,
    out_type=jax.ShapeDtypeStruct((B * 32 // 128, 128), jnp.float32),
    mesh=plsc.VectorSubcoreMesh(core_axis_name="c", subcore_axis_name="s"),
    compiler_params=pltpu.CompilerParams(needs_layout_passes=False),
    scratch_types=[
        pltpu.VMEM((NB,), jnp.int32),
        pltpu.VMEM((NB * S,), jnp.int32),
        pltpu.VMEM((NB * S,), jnp.int32),
        pltpu.VMEM((NB * S,), jnp.int32),
        pltpu.VMEM((NCH * SEG,), jnp.int32),
        pltpu.VMEM((SEG, D), jnp.float32),
        pltpu.VMEM((SEG, D), jnp.float32),
        pltpu.VMEM((16 * 32,), jnp.float32),
        pltpu.VMEM((NB * 32 // 128, 128), jnp.float32),
        pltpu.SemaphoreType.DMA,
        pltpu.SemaphoreType.DMA,
        pltpu.SemaphoreType.DMA,
    ],
)


def _sel_body(pt_ref, nt_ref, po_ref, no_ref):
    ci = lax.broadcasted_iota(jnp.int32, (16, MAX_POS), 1)
    ri = lax.broadcasted_iota(jnp.int32, (16, MAX_POS), 0)
    dn = (((1,), (1,)), ((), ()))
    for ref, cols, out in ((pt_ref, _RP, po_ref), (nt_ref, _RN, no_ref)):
        xi = ref[...]
        tgt = jnp.where(ri == 0, cols[0], 0)
        for s_ in range(1, S):
            tgt = tgt + jnp.where(ri == s_, cols[s_], 0)
        sel_mat = jnp.where(ci == tgt, 1.0, 0.0)
        hi = lax.dot_general(sel_mat, (xi >> 8).astype(jnp.float32), dn,
                             preferred_element_type=jnp.float32)
        lo = lax.dot_general(sel_mat, (xi & 255).astype(jnp.float32), dn,
                             preferred_element_type=jnp.float32)
        out[...] = ((hi + 0.25).astype(jnp.int32) * 256
                    + (lo + 0.25).astype(jnp.int32))


_SELBLK = 4096

_sel = pl.pallas_call(
    _sel_body,
    grid=(pl.cdiv(N_NODES, _SELBLK),),
    in_specs=[pl.BlockSpec((_SELBLK, MAX_POS), lambda i: (i, 0)),
              pl.BlockSpec((_SELBLK, MAX_NEG), lambda i: (i, 0))],
    out_specs=(pl.BlockSpec((16, _SELBLK), lambda i: (0, i)),
               pl.BlockSpec((16, _SELBLK), lambda i: (0, i))),
    out_shape=(jax.ShapeDtypeStruct((16, N_NODES), jnp.int32),
               jax.ShapeDtypeStruct((16, N_NODES), jnp.int32)),
)


def _loss_body(s_ref, o_ref):
    s = s_ref[...]
    col = lax.broadcasted_iota(jnp.int32, s.shape, 1) % 32
    pos_m = col < S
    neg_m = (col >= L) & (col < L + S)
    sp = jnp.log(1.0 + jnp.exp(-jnp.abs(s)))
    softplus_neg_s = jnp.maximum(-s, 0.0) + sp
    softplus_pos_s = jnp.maximum(s, 0.0) + sp
    tot = (jnp.sum(jnp.where(pos_m, softplus_neg_s, 0.0))
           + float(S) * jnp.sum(jnp.where(neg_m, softplus_pos_s, 0.0)))
    o_ref[...] = jnp.reshape(tot / float(B), (1, 1))


_loss = pl.pallas_call(
    _loss_body,
    out_shape=jax.ShapeDtypeStruct((1, 1), jnp.float32),
)


def kernel(node_batch, embeddings, pos_table, neg_table):
    psel, nsel = _sel(pos_table, neg_table)
    scores = _sc_scores(node_batch, embeddings,
                        psel.reshape(-1), nsel.reshape(-1))
    return _loss(scores)[0, 0]


# speedup vs baseline: 1.0631x; 1.0360x over previous
import jax
import jax.numpy as jnp
from jax import lax
from jax.experimental import pallas as pl
from jax.experimental.pallas import tpu as pltpu
from jax.experimental.pallas import tpu_sc as plsc

N_NODES = 50000
D = 512
B = 8192
MAX_POS = 50
MAX_NEG = 50
S = 10

NC, NS, L = 2, 16, 16
NW = NC * NS
NB = B // NW
CH = 2
NCH = NB // CH
SEG = 48
GROWS = SEG
NBUF = 4
NJ = D // L
NKI = (NB * S) // 128

_RP = (31, 5, 40, 13, 47, 18, 43, 36, 22, 17)
_RN = (48, 0, 23, 39, 16, 10, 3, 24, 35, 11)


def _sc_body(nb_hbm, emb_hbm, pt_hbm, nt_hbm, out_hbm,
             nodes_v, tidx, pv2, nv2, aidx, buf0, buf1, buf2, buf3,
             accT, scores_v, semp, sem0, sem1, sem2, sem3):
    wid = lax.axis_index("c") * NS + lax.axis_index("s")
    base = wid * NB
    pltpu.sync_copy(nb_hbm.at[pl.ds(base, NB)], nodes_v)
    lane = lax.iota(jnp.int32, 16)

    for j in range(NB // L):
        nv = nodes_v[pl.ds(j * L, L)]
        n = lane + j * L
        for si in range(S):
            plsc.store_scatter(tidx, [si * NB + n], nv + si * N_NODES)

    def _sid_copies():
        for k in range(NKI):
            isl = tidx.at[pl.ds(k * 128, 128)]
            dst = pl.ds(k * 128, 128)
            yield (pt_hbm.at[isl], pv2.at[dst])
            yield (nt_hbm.at[isl], nv2.at[dst])

    for src, dst in _sid_copies():
        pltpu.async_copy(src, dst, semp)
    for src, dst in _sid_copies():
        pltpu.make_async_copy(src, dst, semp).wait()

    def _rearrange(j, carry):
        n = lane + j * L
        c = n >> 1
        q = n & 1
        bcol = c * SEG + q * S
        for si in range(S):
            plsc.store_scatter(aidx, [bcol + si],
                               pv2[pl.ds(si * NB + j * L, L)])
            plsc.store_scatter(aidx, [bcol + CH * S + si],
                               nv2[pl.ds(si * NB + j * L, L)])
        nv = nodes_v[pl.ds(j * L, L)]
        for w in range(0, SEG - 2 * CH * S, 2):
            plsc.store_scatter(aidx, [c * SEG + 2 * CH * S + w + q], nv)
        return carry

    lax.fori_loop(0, NB // L, _rearrange, 0)

    zero = jnp.zeros((L,), jnp.float32)
    for l in range(16):
        accT[pl.ds(l * 32, 16)] = zero
        accT[pl.ds(l * 32 + 16, 16)] = zero

    def issue(c, buf, sem):
        pltpu.async_copy(emb_hbm.at[aidx.at[pl.ds(c * SEG, GROWS)]], buf, sem)

    def wait_(c, buf, sem):
        pltpu.make_async_copy(emb_hbm.at[aidx.at[pl.ds(c * SEG, GROWS)]], buf, sem).wait()

    def compute(c, buf):
        for q in range(CH):
            nrow = 2 * CH * S + q
            rows = ([q * S + t for t in range(S)]
                    + [CH * S + q * S + t for t in range(S)])

            def body(j, accs):
                o = j * L
                nvec = buf[nrow, pl.ds(o, L)]
                return tuple(a + nvec * buf[r, pl.ds(o, L)]
                             for a, r in zip(accs, rows))

            accs = lax.fori_loop(
                0, NJ, body,
                tuple(jnp.zeros((L,), jnp.float32) for _ in range(2 * S)))
            for t in range(2 * S):
                col = t if t < S else L + (t - S)
                plsc.store_scatter(accT, [lane * 32 + col], accs[t])

            def tree(vs):
                while len(vs) > 1:
                    vs = [a + b for a, b in zip(vs[::2], vs[1::2])]
                return vs[0]

            g = c * CH + q
            scores_v[g >> 2, pl.ds((g & 3) * 32, 16)] = tree(
                [accT[pl.ds(l * 32, 16)] for l in range(16)])
            scores_v[g >> 2, pl.ds((g & 3) * 32 + 16, 16)] = tree(
                [accT[pl.ds(l * 32 + 16, 16)] for l in range(16)])

    bufs = (buf0, buf1, buf2, buf3)
    sems = (sem0, sem1, sem2, sem3)
    for d in range(NBUF):
        issue(d, bufs[d], sems[d])

    def iter_body(i, carry):
        for d in range(NBUF):
            k = NBUF * i + d
            wait_(k, bufs[d], sems[d])
            compute(k, bufs[d])

            @pl.when(i < NCH // NBUF - 1)
            def _():
                issue(k + NBUF, bufs[d], sems[d])
        return carry

    lax.fori_loop(0, NCH // NBUF, iter_body, 0)
    pltpu.sync_copy(scores_v, out_hbm.at[pl.ds(wid * (NB * 32 // 128), NB * 32 // 128)])


_sc_scores = pl.kernel(
    _sc_body,
    out_type=jax.ShapeDtypeStruct((B * 32 // 128, 128), jnp.float32),
    mesh=plsc.VectorSubcoreMesh(core_axis_name="c", subcore_axis_name="s"),
    compiler_params=pltpu.CompilerParams(needs_layout_passes=False),
    scratch_types=[
        pltpu.VMEM((NB,), jnp.int32),
        pltpu.VMEM((NB * S,), jnp.int32),
        pltpu.VMEM((NB * S,), jnp.int32),
        pltpu.VMEM((NB * S,), jnp.int32),
        pltpu.VMEM((NCH * SEG,), jnp.int32),
        pltpu.VMEM((GROWS, D), jnp.float32),
        pltpu.VMEM((GROWS, D), jnp.float32),
        pltpu.VMEM((GROWS, D), jnp.float32),
        pltpu.VMEM((GROWS, D), jnp.float32),
        pltpu.VMEM((16 * 32,), jnp.float32),
        pltpu.VMEM((NB * 32 // 128, 128), jnp.float32),
        pltpu.SemaphoreType.DMA,
        pltpu.SemaphoreType.DMA,
        pltpu.SemaphoreType.DMA,
        pltpu.SemaphoreType.DMA,
        pltpu.SemaphoreType.DMA,
    ],
)


def _sel_body(pt_ref, nt_ref, po_ref, no_ref):
    ci = lax.broadcasted_iota(jnp.int32, (16, MAX_POS), 1)
    ri = lax.broadcasted_iota(jnp.int32, (16, MAX_POS), 0)
    dn = (((1,), (1,)), ((), ()))
    for ref, cols, out in ((pt_ref, _RP, po_ref), (nt_ref, _RN, no_ref)):
        xi = ref[...]
        tgt = jnp.where(ri == 0, cols[0], 0)
        for s_ in range(1, S):
            tgt = tgt + jnp.where(ri == s_, cols[s_], 0)
        sel_mat = jnp.where(ci == tgt, 1.0, 0.0)
        hi = lax.dot_general(sel_mat, (xi >> 8).astype(jnp.float32), dn,
                             preferred_element_type=jnp.float32)
        lo = lax.dot_general(sel_mat, (xi & 255).astype(jnp.float32), dn,
                             preferred_element_type=jnp.float32)
        out[...] = ((hi + 0.25).astype(jnp.int32) * 256
                    + (lo + 0.25).astype(jnp.int32))


_SELBLK = 4096

_sel = pl.pallas_call(
    _sel_body,
    grid=(pl.cdiv(N_NODES, _SELBLK),),
    in_specs=[pl.BlockSpec((_SELBLK, MAX_POS), lambda i: (i, 0)),
              pl.BlockSpec((_SELBLK, MAX_NEG), lambda i: (i, 0))],
    out_specs=(pl.BlockSpec((16, _SELBLK), lambda i: (0, i)),
               pl.BlockSpec((16, _SELBLK), lambda i: (0, i))),
    out_shape=(jax.ShapeDtypeStruct((16, N_NODES), jnp.int32),
               jax.ShapeDtypeStruct((16, N_NODES), jnp.int32)),
)


def _loss_body(s_ref, o_ref):
    s = s_ref[...]
    col = lax.broadcasted_iota(jnp.int32, s.shape, 1) % 32
    pos_m = col < S
    neg_m = (col >= L) & (col < L + S)
    sp = jnp.log(1.0 + jnp.exp(-jnp.abs(s)))
    softplus_neg_s = jnp.maximum(-s, 0.0) + sp
    softplus_pos_s = jnp.maximum(s, 0.0) + sp
    tot = (jnp.sum(jnp.where(pos_m, softplus_neg_s, 0.0))
           + float(S) * jnp.sum(jnp.where(neg_m, softplus_pos_s, 0.0)))
    o_ref[...] = jnp.reshape(tot / float(B), (1, 1))


_loss = pl.pallas_call(
    _loss_body,
    out_shape=jax.ShapeDtypeStruct((1, 1), jnp.float32),
)


def kernel(node_batch, embeddings, pos_table, neg_table):
    psel, nsel = _sel(pos_table, neg_table)
    scores = _sc_scores(node_batch, embeddings,
                        psel.reshape(-1), nsel.reshape(-1))
    return _loss(scores)[0, 0]


# speedup vs baseline: 1.0652x; 1.0020x over previous
import jax
import jax.numpy as jnp
from jax import lax
from jax.experimental import pallas as pl
from jax.experimental.pallas import tpu as pltpu
from jax.experimental.pallas import tpu_sc as plsc

N_NODES = 50000
D = 512
B = 8192
MAX_POS = 50
MAX_NEG = 50
S = 10

NC, NS, L = 2, 16, 16
NW = NC * NS
NB = B // NW
CH = 2
NCH = NB // CH
SEG = 48
GROWS = 44
NBUF = 4
NJ = D // L
NKI = (NB * S) // 128

_RP = (31, 5, 40, 13, 47, 18, 43, 36, 22, 17)
_RN = (48, 0, 23, 39, 16, 10, 3, 24, 35, 11)


def _sc_body(nb_hbm, emb_hbm, pt_hbm, nt_hbm, out_hbm,
             nodes_v, tidx, pv2, nv2, aidx, buf0, buf1, buf2, buf3,
             accT, scores_v, semp, sem0, sem1, sem2, sem3):
    wid = lax.axis_index("c") * NS + lax.axis_index("s")
    base = wid * NB
    pltpu.sync_copy(nb_hbm.at[pl.ds(base, NB)], nodes_v)
    lane = lax.iota(jnp.int32, 16)

    for j in range(NB // L):
        nv = nodes_v[pl.ds(j * L, L)]
        n = lane + j * L
        for si in range(S):
            plsc.store_scatter(tidx, [si * NB + n], nv + si * N_NODES)

    def _sid_copies():
        for k in range(NKI):
            isl = tidx.at[pl.ds(k * 128, 128)]
            dst = pl.ds(k * 128, 128)
            yield (pt_hbm.at[isl], pv2.at[dst])
            yield (nt_hbm.at[isl], nv2.at[dst])

    for src, dst in _sid_copies():
        pltpu.async_copy(src, dst, semp)
    for src, dst in _sid_copies():
        pltpu.make_async_copy(src, dst, semp).wait()

    def _rearrange(j, carry):
        n = lane + j * L
        c = n >> 1
        q = n & 1
        bcol = c * SEG + q * S
        for si in range(S):
            plsc.store_scatter(aidx, [bcol + si],
                               pv2[pl.ds(si * NB + j * L, L)])
            plsc.store_scatter(aidx, [bcol + CH * S + si],
                               nv2[pl.ds(si * NB + j * L, L)])
        nv = nodes_v[pl.ds(j * L, L)]
        for w in range(0, GROWS - 2 * CH * S, 2):
            plsc.store_scatter(aidx, [c * SEG + 2 * CH * S + w + q], nv)
        return carry

    lax.fori_loop(0, NB // L, _rearrange, 0)

    zero = jnp.zeros((L,), jnp.float32)
    for l in range(16):
        accT[pl.ds(l * 32, 16)] = zero
        accT[pl.ds(l * 32 + 16, 16)] = zero

    def issue(c, buf, sem):
        pltpu.async_copy(emb_hbm.at[aidx.at[pl.ds(c * SEG, GROWS)]], buf, sem)

    def wait_(c, buf, sem):
        pltpu.make_async_copy(emb_hbm.at[aidx.at[pl.ds(c * SEG, GROWS)]], buf, sem).wait()

    def compute(c, buf):
        for q in range(CH):
            nrow = 2 * CH * S + q
            rows = ([q * S + t for t in range(S)]
                    + [CH * S + q * S + t for t in range(S)])

            def body(j, accs):
                o = j * L
                nvec = buf[nrow, pl.ds(o, L)]
                return tuple(a + nvec * buf[r, pl.ds(o, L)]
                             for a, r in zip(accs, rows))

            accs = lax.fori_loop(
                0, NJ, body,
                tuple(jnp.zeros((L,), jnp.float32) for _ in range(2 * S)))
            for t in range(2 * S):
                col = t if t < S else L + (t - S)
                plsc.store_scatter(accT, [lane * 32 + col], accs[t])

            def tree(vs):
                while len(vs) > 1:
                    vs = [a + b for a, b in zip(vs[::2], vs[1::2])]
                return vs[0]

            g = c * CH + q
            scores_v[g >> 2, pl.ds((g & 3) * 32, 16)] = tree(
                [accT[pl.ds(l * 32, 16)] for l in range(16)])
            scores_v[g >> 2, pl.ds((g & 3) * 32 + 16, 16)] = tree(
                [accT[pl.ds(l * 32 + 16, 16)] for l in range(16)])

    bufs = (buf0, buf1, buf2, buf3)
    sems = (sem0, sem1, sem2, sem3)
    for d in range(NBUF):
        issue(d, bufs[d], sems[d])

    def iter_body(i, carry):
        for d in range(NBUF):
            k = NBUF * i + d
            wait_(k, bufs[d], sems[d])
            compute(k, bufs[d])

            @pl.when(i < NCH // NBUF - 1)
            def _():
                issue(k + NBUF, bufs[d], sems[d])
        return carry

    lax.fori_loop(0, NCH // NBUF, iter_body, 0)
    pltpu.sync_copy(scores_v, out_hbm.at[pl.ds(wid * (NB * 32 // 128), NB * 32 // 128)])


_sc_scores = pl.kernel(
    _sc_body,
    out_type=jax.ShapeDtypeStruct((B * 32 // 128, 128), jnp.float32),
    mesh=plsc.VectorSubcoreMesh(core_axis_name="c", subcore_axis_name="s"),
    compiler_params=pltpu.CompilerParams(needs_layout_passes=False),
    scratch_types=[
        pltpu.VMEM((NB,), jnp.int32),
        pltpu.VMEM((NB * S,), jnp.int32),
        pltpu.VMEM((NB * S,), jnp.int32),
        pltpu.VMEM((NB * S,), jnp.int32),
        pltpu.VMEM((NCH * SEG,), jnp.int32),
        pltpu.VMEM((GROWS, D), jnp.float32),
        pltpu.VMEM((GROWS, D), jnp.float32),
        pltpu.VMEM((GROWS, D), jnp.float32),
        pltpu.VMEM((GROWS, D), jnp.float32),
        pltpu.VMEM((16 * 32,), jnp.float32),
        pltpu.VMEM((NB * 32 // 128, 128), jnp.float32),
        pltpu.SemaphoreType.DMA,
        pltpu.SemaphoreType.DMA,
        pltpu.SemaphoreType.DMA,
        pltpu.SemaphoreType.DMA,
        pltpu.SemaphoreType.DMA,
    ],
)


def _sel_body(pt_ref, nt_ref, po_ref, no_ref):
    ci = lax.broadcasted_iota(jnp.int32, (16, MAX_POS), 1)
    ri = lax.broadcasted_iota(jnp.int32, (16, MAX_POS), 0)
    dn = (((1,), (1,)), ((), ()))
    for ref, cols, out in ((pt_ref, _RP, po_ref), (nt_ref, _RN, no_ref)):
        xi = ref[...]
        tgt = jnp.where(ri == 0, cols[0], 0)
        for s_ in range(1, S):
            tgt = tgt + jnp.where(ri == s_, cols[s_], 0)
        sel_mat = jnp.where(ci == tgt, 1.0, 0.0)
        hi = lax.dot_general(sel_mat, (xi >> 8).astype(jnp.float32), dn,
                             preferred_element_type=jnp.float32)
        lo = lax.dot_general(sel_mat, (xi & 255).astype(jnp.float32), dn,
                             preferred_element_type=jnp.float32)
        out[...] = ((hi + 0.25).astype(jnp.int32) * 256
                    + (lo + 0.25).astype(jnp.int32))


_SELBLK = 4096

_sel = pl.pallas_call(
    _sel_body,
    grid=(pl.cdiv(N_NODES, _SELBLK),),
    in_specs=[pl.BlockSpec((_SELBLK, MAX_POS), lambda i: (i, 0)),
              pl.BlockSpec((_SELBLK, MAX_NEG), lambda i: (i, 0))],
    out_specs=(pl.BlockSpec((16, _SELBLK), lambda i: (0, i)),
               pl.BlockSpec((16, _SELBLK), lambda i: (0, i))),
    out_shape=(jax.ShapeDtypeStruct((16, N_NODES), jnp.int32),
               jax.ShapeDtypeStruct((16, N_NODES), jnp.int32)),
)


def _loss_body(s_ref, o_ref):
    s = s_ref[...]
    col = lax.broadcasted_iota(jnp.int32, s.shape, 1) % 32
    pos_m = col < S
    neg_m = (col >= L) & (col < L + S)
    sp = jnp.log(1.0 + jnp.exp(-jnp.abs(s)))
    softplus_neg_s = jnp.maximum(-s, 0.0) + sp
    softplus_pos_s = jnp.maximum(s, 0.0) + sp
    tot = (jnp.sum(jnp.where(pos_m, softplus_neg_s, 0.0))
           + float(S) * jnp.sum(jnp.where(neg_m, softplus_pos_s, 0.0)))
    o_ref[...] = jnp.reshape(tot / float(B), (1, 1))


_loss = pl.pallas_call(
    _loss_body,
    out_shape=jax.ShapeDtypeStruct((1, 1), jnp.float32),
)


def kernel(node_batch, embeddings, pos_table, neg_table):
    psel, nsel = _sel(pos_table, neg_table)
    scores = _sc_scores(node_batch, embeddings,
                        psel.reshape(-1), nsel.reshape(-1))
    return _loss(scores)[0, 0]
